# Initial kernel scaffold; baseline (speedup 1.0000x reference)
import jax
import jax.numpy as jnp
from jax.experimental import pallas as pl


def kernel(edge_index, h, x, edge_attr, node_mask, edge_mask, W_e1, b_e1, W_e2, b_e2, W_att, b_att, W_h1, b_h1, W_h2, b_h2, W_x1, b_x1, W_x2, b_x2):
    raise NotImplementedError("write your pallas kernel here")


# speedup vs baseline: 3.4956x; 3.4956x over previous
import functools
import jax
import jax.numpy as jnp
from jax import lax
from jax.experimental import pallas as pl
from jax.experimental.pallas import tpu as pltpu
from jax.experimental.pallas import tpu_sc as plsc

N = 10000
E = 320000
D = 128
DE = 16

NC = 2
NS = 16
NW = NC * NS
EPW = E // NW
K = 80
NCH = EPW // K
ROWS = N // NS
ZR = 125

_mesh = plsc.VectorSubcoreMesh(core_axis_name="c", subcore_axis_name="s")
_sc_params = pltpu.CompilerParams(use_tc_tiling_on_sc=False)


def _pre_body(h_ref, wi_ref, wj_ref, wh_ref, hs_ref, hr_ref, hh_ref):
    hb = h_ref[...]
    hs_ref[...] = jnp.dot(hb, wi_ref[...], preferred_element_type=jnp.float32)
    hr_ref[...] = jnp.dot(hb, wj_ref[...], preferred_element_type=jnp.float32)
    hh_ref[...] = jnp.dot(hb, wh_ref[...], preferred_element_type=jnp.float32)


def _precompute(h, w1hi, w1hj, wh1a):
    bn = 2000
    blk = pl.BlockSpec((bn, D), lambda i: (i, 0))
    wblk = pl.BlockSpec((D, D), lambda i: (0, 0))
    out = jax.ShapeDtypeStruct((N, D), jnp.float32)
    return pl.pallas_call(
        _pre_body,
        grid=(N // bn,),
        in_specs=[blk, wblk, wblk, wblk],
        out_specs=[blk, blk, blk],
        out_shape=[out, out, out],
    )(h, w1hi, w1hj, wh1a)


def _sc_gather_body(hs_hbm, hr_hbm, xp_hbm, s_hbm, r_hbm, w1r_hbm,
                    g_hbm, xd_hbm,
                    sidx, ridx, bufa, bufb, bufxs, bufxr, w1rv,
                    sem1, sem2, sem3, sem4):
    c = lax.axis_index("c")
    s = lax.axis_index("s")
    wid = c * NS + s
    base0 = wid * EPW
    pltpu.sync_copy(w1r_hbm, w1rv)

    def chunk(i, carry):
        base = pl.multiple_of(base0 + i * K, 8)
        pltpu.sync_copy(s_hbm.at[pl.ds(base, K)], sidx)
        pltpu.sync_copy(r_hbm.at[pl.ds(base, K)], ridx)
        cpa = pltpu.async_copy(hs_hbm.at[sidx], bufa, sem1)
        cpb = pltpu.async_copy(hr_hbm.at[ridx], bufb, sem2)
        cpxs = pltpu.async_copy(xp_hbm.at[sidx], bufxs, sem3)
        cpxr = pltpu.async_copy(xp_hbm.at[ridx], bufxr, sem4)
        cpxs.wait()
        cpxr.wait()
        cpa.wait()
        cpb.wait()

        def row(e, cr):
            v = bufxs[e, :] - bufxr[e, :]
            bufxs[e, :] = v
            sq = v * v
            rad = sq[0] + sq[1] + sq[2]
            for j in range(D // 16):
                sl = pl.ds(j * 16, 16)
                bufa[e, sl] = bufa[e, sl] + bufb[e, sl] + rad * w1rv[sl]
            return cr

        lax.fori_loop(0, K, row, 0)
        pltpu.sync_copy(bufa, g_hbm.at[pl.ds(base, K)])
        pltpu.sync_copy(bufxs, xd_hbm.at[pl.ds(base, K)])
        return carry

    lax.fori_loop(0, NCH, chunk, 0)


_sc_gather = functools.partial(
    pl.kernel,
    mesh=_mesh,
    out_type=[jax.ShapeDtypeStruct((E, D), jnp.float32),
              jax.ShapeDtypeStruct((E, 16), jnp.float32)],
    scratch_types=[pltpu.VMEM((K,), jnp.int32),
                   pltpu.VMEM((K,), jnp.int32),
                   pltpu.VMEM((K, D), jnp.float32),
                   pltpu.VMEM((K, D), jnp.float32),
                   pltpu.VMEM((K, 16), jnp.float32),
                   pltpu.VMEM((K, 16), jnp.float32),
                   pltpu.VMEM((D,), jnp.float32),
                   pltpu.SemaphoreType.DMA,
                   pltpu.SemaphoreType.DMA,
                   pltpu.SemaphoreType.DMA,
                   pltpu.SemaphoreType.DMA],
    compiler_params=_sc_params,
)(_sc_gather_body)


BE = 2560


def _edge_body(g_ref, ea_ref, em_ref, we2_ref, wx1_ref, wea_ref,
               aux_ref, m_ref, p_ref):
    g = g_ref[...]
    aux = aux_ref[...]
    be1 = aux[1:2, :]
    be2 = aux[2:3, :]
    bx1 = aux[3:4, :]
    watt = aux[5:6, :]
    wx2 = aux[6:7, :]
    batt = aux[4, 0]
    bx2 = aux[4, 1]

    t1 = g + jnp.dot(ea_ref[...], wea_ref[...],
                     preferred_element_type=jnp.float32) + be1
    o1 = t1 * jax.nn.sigmoid(t1)
    t2 = jnp.dot(o1, we2_ref[...], preferred_element_type=jnp.float32) + be2
    o2 = t2 * jax.nn.sigmoid(t2)
    att = jax.nn.sigmoid(jnp.sum(o2 * watt, axis=1, keepdims=True) + batt)
    m = o2 * att * em_ref[...]
    m_ref[...] = m
    t3 = jnp.dot(m, wx1_ref[...], preferred_element_type=jnp.float32) + bx1
    h3 = t3 * jax.nn.sigmoid(t3)
    p = jnp.sum(h3 * wx2, axis=1, keepdims=True) + bx2
    p_ref[...] = p.reshape(1, BE // 128, 128)


def _edge_mlp(g, ea, em, we2, wx1, wea, aux):
    return pl.pallas_call(
        _edge_body,
        grid=(E // BE,),
        in_specs=[pl.BlockSpec((BE, D), lambda i: (i, 0)),
                  pl.BlockSpec((BE, DE), lambda i: (i, 0)),
                  pl.BlockSpec((BE, 1), lambda i: (i, 0)),
                  pl.BlockSpec((D, D), lambda i: (0, 0)),
                  pl.BlockSpec((D, D), lambda i: (0, 0)),
                  pl.BlockSpec((DE, D), lambda i: (0, 0)),
                  pl.BlockSpec((8, D), lambda i: (0, 0))],
        out_specs=[pl.BlockSpec((BE, D), lambda i: (i, 0)),
                   pl.BlockSpec((1, BE // 128, 128), lambda i: (i, 0, 0))],
        out_shape=[jax.ShapeDtypeStruct((E, D), jnp.float32),
                   jax.ShapeDtypeStruct((E // BE, BE // 128, 128), jnp.float32)],
    )(g, ea, em, we2, wx1, wea, aux)


def _sc_scatter_body(s_hbm, m_hbm, xd_hbm, p_hbm, mi_hbm, xa_hbm,
                     sidx, mbuf, xdbuf, pbuf, zbuf, zbufx, accm, accx,
                     sem1, sem2):
    c = lax.axis_index("c")
    s = lax.axis_index("s")
    wid = c * NS + s

    def zrow(e, cr):
        for j in range(D // 16):
            zbuf[e, pl.ds(j * 16, 16)] = jnp.zeros((16,), jnp.float32)
        zbufx[e, :] = jnp.zeros((16,), jnp.float32)
        return cr

    lax.fori_loop(0, ZR, zrow, 0)

    def zcp(t, cr):
        rb = s * ROWS + t * ZR
        pltpu.sync_copy(zbuf, accm.at[pl.ds(rb, ZR)])
        pltpu.sync_copy(zbufx, accx.at[pl.ds(rb, ZR)])
        return cr

    lax.fori_loop(0, ROWS // ZR, zcp, 0)
    plsc.subcore_barrier()

    iota16 = lax.iota(jnp.int32, 16)
    e3v = jnp.where(iota16 == 3, 1.0, 0.0).astype(jnp.float32)
    base0 = wid * EPW

    def chunk(i, carry):
        base = pl.multiple_of(base0 + i * K, 8)
        pltpu.sync_copy(s_hbm.at[pl.ds(base, K)], sidx)
        pltpu.sync_copy(p_hbm.at[pl.ds(base, K)], pbuf)
        cpm = pltpu.async_copy(m_hbm.at[pl.ds(base, K)], mbuf, sem1)
        cpx = pltpu.async_copy(xd_hbm.at[pl.ds(base, K)], xdbuf, sem2)
        cpx.wait()

        def grp(g, cr):
            pv = pbuf[pl.ds(g * 16, 16)]
            for l in range(16):
                e = g * 16 + l
                xdbuf[e, :] = xdbuf[e, :] * pv[l] + e3v
            return cr

        lax.fori_loop(0, K // 16, grp, 0)
        pltpu.sync_copy(xdbuf, accx.at[sidx], add=True)
        cpm.wait()
        pltpu.sync_copy(mbuf, accm.at[sidx], add=True)
        return carry

    lax.fori_loop(0, NCH, chunk, 0)
    plsc.subcore_barrier()

    def ocp(t, cr):
        rb = s * ROWS + t * ZR
        pltpu.sync_copy(accm.at[pl.ds(rb, ZR)], mi_hbm.at[c, pl.ds(rb, ZR)])
        pltpu.sync_copy(accx.at[pl.ds(rb, ZR)], xa_hbm.at[c, pl.ds(rb, ZR)])
        return cr

    lax.fori_loop(0, ROWS // ZR, ocp, 0)


_sc_scatter = functools.partial(
    pl.kernel,
    mesh=_mesh,
    out_type=[jax.ShapeDtypeStruct((NC, N, D), jnp.float32),
              jax.ShapeDtypeStruct((NC, N, 16), jnp.float32)],
    scratch_types=[pltpu.VMEM((K,), jnp.int32),
                   pltpu.VMEM((K, D), jnp.float32),
                   pltpu.VMEM((K, 16), jnp.float32),
                   pltpu.VMEM((K,), jnp.float32),
                   pltpu.VMEM((ZR, D), jnp.float32),
                   pltpu.VMEM((ZR, 16), jnp.float32),
                   pltpu.VMEM_SHARED((N, D), jnp.float32),
                   pltpu.VMEM_SHARED((N, 16), jnp.float32),
                   pltpu.SemaphoreType.DMA,
                   pltpu.SemaphoreType.DMA],
    compiler_params=_sc_params,
)(_sc_scatter_body)


def _node_body(h_ref, hh_ref, mi_ref, xa_ref, xp_ref, nm_ref, wh1b_ref,
               wh2_ref, aux_ref, hn_ref, co_ref):
    aux = aux_ref[...]
    bh1 = aux[0:1, :]
    bh2 = aux[1:2, :]
    m2 = mi_ref[...]
    mi = m2[0] + m2[1]
    x2 = xa_ref[...]
    xa = x2[0] + x2[1]
    nm = nm_ref[...]
    t = hh_ref[...] + jnp.dot(mi, wh1b_ref[...],
                              preferred_element_type=jnp.float32) + bh1
    u = t * jax.nn.sigmoid(t)
    hn = h_ref[...] + jnp.dot(u, wh2_ref[...],
                              preferred_element_type=jnp.float32) + bh2
    hn_ref[...] = hn * nm
    cnt = xa[:, 3:4]
    mean = xa[:, 0:3] / cnt
    co_ref[...] = (xp_ref[...][:, 0:3] + mean) * nm


def _node_update(h, hh, mi, xa, xp, nm, wh1b, wh2, aux2):
    bn = 2000
    blk = pl.BlockSpec((bn, D), lambda i: (i, 0))
    return pl.pallas_call(
        _node_body,
        grid=(N // bn,),
        in_specs=[blk, blk,
                  pl.BlockSpec((NC, bn, D), lambda i: (0, i, 0)),
                  pl.BlockSpec((NC, bn, 16), lambda i: (0, i, 0)),
                  pl.BlockSpec((bn, 16), lambda i: (i, 0)),
                  pl.BlockSpec((bn, 1), lambda i: (i, 0)),
                  pl.BlockSpec((D, D), lambda i: (0, 0)),
                  pl.BlockSpec((D, D), lambda i: (0, 0)),
                  pl.BlockSpec((2, D), lambda i: (0, 0))],
        out_specs=[blk, pl.BlockSpec((bn, 3), lambda i: (i, 0))],
        out_shape=[jax.ShapeDtypeStruct((N, D), jnp.float32),
                   jax.ShapeDtypeStruct((N, 3), jnp.float32)],
    )(h, hh, mi, xa, xp, nm, wh1b, wh2, aux2)


def kernel(edge_index, h, x, edge_attr, node_mask, edge_mask,
           W_e1, b_e1, W_e2, b_e2, W_att, b_att,
           W_h1, b_h1, W_h2, b_h2, W_x1, b_x1, W_x2, b_x2):
    w1hi = W_e1[:D]
    w1hj = W_e1[D:2 * D]
    w1r = W_e1[2 * D]
    wea = W_e1[2 * D + 1:]
    wh1a = W_h1[:D]
    wh1b = W_h1[D:]
    senders = edge_index[0]
    receivers = edge_index[1]
    xpad = jnp.pad(x, ((0, 0), (0, 13)))

    scal = jnp.zeros((D,), jnp.float32).at[0].set(b_att[0]).at[1].set(b_x2[0])
    aux = jnp.stack([w1r, b_e1, b_e2, b_x1, scal,
                     W_att[:, 0], W_x2[:, 0], jnp.zeros((D,), jnp.float32)])
    aux2 = jnp.stack([b_h1, b_h2])

    hs, hr, hh = _precompute(h, w1hi, w1hj, wh1a)
    g, xd = _sc_gather(hs, hr, xpad, senders, receivers, w1r)
    m, p2d = _edge_mlp(g, edge_attr, edge_mask, W_e2, W_x1, wea, aux)
    pflat = p2d.reshape(E)
    mi, xa = _sc_scatter(senders, m, xd, pflat)
    h_new, coord = _node_update(h, hh, mi, xa, xpad, node_mask, wh1b, W_h2, aux2)
    return (h_new, coord)


# speedup vs baseline: 4.8897x; 1.3988x over previous
import functools
import jax
import jax.numpy as jnp
from jax import lax
from jax.experimental import pallas as pl
from jax.experimental.pallas import tpu as pltpu
from jax.experimental.pallas import tpu_sc as plsc

N = 10000
E = 320000
D = 128
DE = 16

NC = 2
NS = 16
NW = NC * NS
EPW = E // NW
KC = 128
NF = EPW // KC
KT = EPW - NF * KC
ROWS = N // NS
ZR = 25

_mesh = plsc.VectorSubcoreMesh(core_axis_name="c", subcore_axis_name="s")
_sc_params = pltpu.CompilerParams(use_tc_tiling_on_sc=False)


def _pre_body(h_ref, wi_ref, wj_ref, wh_ref, hs_ref, hr_ref, hh_ref):
    hb = h_ref[...]
    hs_ref[...] = jnp.dot(hb, wi_ref[...], preferred_element_type=jnp.float32)
    hr_ref[...] = jnp.dot(hb, wj_ref[...], preferred_element_type=jnp.float32)
    hh_ref[...] = jnp.dot(hb, wh_ref[...], preferred_element_type=jnp.float32)


def _precompute(h, w1hi, w1hj, wh1a):
    bn = 2000
    blk = pl.BlockSpec((bn, D), lambda i: (i, 0))
    wblk = pl.BlockSpec((D, D), lambda i: (0, 0))
    out = jax.ShapeDtypeStruct((N, D), jnp.float32)
    return pl.pallas_call(
        _pre_body,
        grid=(N // bn,),
        in_specs=[blk, wblk, wblk, wblk],
        out_specs=[blk, blk, blk],
        out_shape=[out, out, out],
    )(h, w1hi, w1hj, wh1a)


def _sc_gather_body(hs_hbm, hr_hbm, xp_hbm, s_hbm, r_hbm, w1r_hbm,
                    g_hbm, xd_hbm,
                    sall, rall, w1rv,
                    bufa0, bufb0, bufxs0, bufxr0,
                    bufa1, bufb1, bufxs1, bufxr1,
                    gsem0, gsem1, wsem0, wsem1):
    c = lax.axis_index("c")
    s = lax.axis_index("s")
    wid = c * NS + s
    base0 = wid * EPW
    pltpu.sync_copy(w1r_hbm, w1rv)
    pltpu.sync_copy(s_hbm.at[pl.ds(base0, EPW)], sall)
    pltpu.sync_copy(r_hbm.at[pl.ds(base0, EPW)], rall)

    sets = ((bufa0, bufb0, bufxs0, bufxr0, gsem0, wsem0),
            (bufa1, bufb1, bufxs1, bufxr1, gsem1, wsem1))

    def start_gathers(i, b):
        ba, bb, bxs, bxr, gs, _ = sets[b]
        off = pl.multiple_of(i * KC, 8)
        si = sall.at[pl.ds(off, KC)]
        ri = rall.at[pl.ds(off, KC)]
        pltpu.async_copy(hs_hbm.at[si], ba, gs)
        pltpu.async_copy(hr_hbm.at[ri], bb, gs)
        pltpu.async_copy(xp_hbm.at[si], bxs, gs)
        pltpu.async_copy(xp_hbm.at[ri], bxr, gs)

    def wait_gathers(b):
        ba, bb, bxs, bxr, gs, _ = sets[b]
        pltpu.make_async_copy(hs_hbm.at[pl.ds(0, KC)], ba, gs).wait()
        pltpu.make_async_copy(hr_hbm.at[pl.ds(0, KC)], bb, gs).wait()
        pltpu.make_async_copy(xp_hbm.at[pl.ds(0, KC)], bxs, gs).wait()
        pltpu.make_async_copy(xp_hbm.at[pl.ds(0, KC)], bxr, gs).wait()

    def compute(b, nrows):
        ba, bb, bxs, bxr, _, _ = sets[b]

        def row(e, cr):
            v = bxs[e, :] - bxr[e, :]
            bxs[e, :] = v
            sq = v * v
            rad = sq[0] + sq[1] + sq[2]
            for j in range(D // 16):
                sl = pl.ds(j * 16, 16)
                ba[e, sl] = ba[e, sl] + bb[e, sl] + rad * w1rv[sl]
            return cr

        lax.fori_loop(0, nrows, row, 0)

    def start_writes(i, b):
        ba, _, bxs, _, _, ws = sets[b]
        base = pl.multiple_of(base0 + i * KC, 8)
        pltpu.async_copy(ba, g_hbm.at[pl.ds(base, KC)], ws)
        pltpu.async_copy(bxs, xd_hbm.at[pl.ds(base, KC)], ws)

    def wait_writes(b):
        ba, _, bxs, _, _, ws = sets[b]
        pltpu.make_async_copy(ba, g_hbm.at[pl.ds(0, KC)], ws).wait()
        pltpu.make_async_copy(bxs, xd_hbm.at[pl.ds(0, KC)], ws).wait()

    start_gathers(0, 0)

    def pair(t, carry):
        wait_gathers(0)

        @pl.when(t > 0)
        def _():
            wait_writes(1)

        start_gathers(2 * t + 1, 1)
        compute(0, KC)
        start_writes(2 * t, 0)

        wait_gathers(1)

        @pl.when(t < NF // 2 - 1)
        def _():
            wait_writes(0)
            start_gathers(2 * t + 2, 0)

        compute(1, KC)
        start_writes(2 * t + 1, 1)
        return carry

    lax.fori_loop(0, NF // 2, pair, 0)

    ba, bb, bxs, bxr, gs, ws = sets[0]
    toff = NF * KC
    sv = sall[pl.ds(toff, KT)]
    rv = rall[pl.ds(toff, KT)]
    wait_writes(0)
    pltpu.async_copy(hs_hbm.at[sv], ba.at[pl.ds(0, KT)], gs)
    pltpu.async_copy(hr_hbm.at[rv], bb.at[pl.ds(0, KT)], gs)
    pltpu.async_copy(xp_hbm.at[sv], bxs.at[pl.ds(0, KT)], gs)
    pltpu.async_copy(xp_hbm.at[rv], bxr.at[pl.ds(0, KT)], gs)
    pltpu.make_async_copy(hs_hbm.at[pl.ds(0, KT)], ba.at[pl.ds(0, KT)], gs).wait()
    pltpu.make_async_copy(hr_hbm.at[pl.ds(0, KT)], bb.at[pl.ds(0, KT)], gs).wait()
    pltpu.make_async_copy(xp_hbm.at[pl.ds(0, KT)], bxs.at[pl.ds(0, KT)], gs).wait()
    pltpu.make_async_copy(xp_hbm.at[pl.ds(0, KT)], bxr.at[pl.ds(0, KT)], gs).wait()
    compute(0, KT)
    tb = pl.multiple_of(base0 + toff, 8)
    pltpu.async_copy(ba.at[pl.ds(0, KT)], g_hbm.at[pl.ds(tb, KT)], ws)
    pltpu.async_copy(bxs.at[pl.ds(0, KT)], xd_hbm.at[pl.ds(tb, KT)], ws)
    wait_writes(1)
    pltpu.make_async_copy(ba.at[pl.ds(0, KT)], g_hbm.at[pl.ds(0, KT)], ws).wait()
    pltpu.make_async_copy(bxs.at[pl.ds(0, KT)], xd_hbm.at[pl.ds(0, KT)], ws).wait()


_sc_gather = functools.partial(
    pl.kernel,
    mesh=_mesh,
    out_type=[jax.ShapeDtypeStruct((E, D), jnp.float32),
              jax.ShapeDtypeStruct((E, 16), jnp.float32)],
    scratch_types=[pltpu.VMEM((EPW,), jnp.int32),
                   pltpu.VMEM((EPW,), jnp.int32),
                   pltpu.VMEM((D,), jnp.float32),
                   pltpu.VMEM((KC, D), jnp.float32),
                   pltpu.VMEM((KC, D), jnp.float32),
                   pltpu.VMEM((KC, 16), jnp.float32),
                   pltpu.VMEM((KC, 16), jnp.float32),
                   pltpu.VMEM((KC, D), jnp.float32),
                   pltpu.VMEM((KC, D), jnp.float32),
                   pltpu.VMEM((KC, 16), jnp.float32),
                   pltpu.VMEM((KC, 16), jnp.float32),
                   pltpu.SemaphoreType.DMA,
                   pltpu.SemaphoreType.DMA,
                   pltpu.SemaphoreType.DMA,
                   pltpu.SemaphoreType.DMA],
    compiler_params=_sc_params,
)(_sc_gather_body)


BE = 2560


def _edge_body(g_ref, ea_ref, em_ref, we2_ref, wx1_ref, wea_ref,
               aux_ref, m_ref, p_ref):
    g = g_ref[...]
    aux = aux_ref[...]
    be1 = aux[1:2, :]
    be2 = aux[2:3, :]
    bx1 = aux[3:4, :]
    watt = aux[5:6, :]
    wx2 = aux[6:7, :]
    batt = aux[4, 0]
    bx2 = aux[4, 1]

    t1 = g + jnp.dot(ea_ref[...], wea_ref[...],
                     preferred_element_type=jnp.float32) + be1
    o1 = t1 * jax.nn.sigmoid(t1)
    t2 = jnp.dot(o1, we2_ref[...], preferred_element_type=jnp.float32) + be2
    o2 = t2 * jax.nn.sigmoid(t2)
    att = jax.nn.sigmoid(jnp.sum(o2 * watt, axis=1, keepdims=True) + batt)
    m = o2 * att * em_ref[...]
    m_ref[...] = m
    t3 = jnp.dot(m, wx1_ref[...], preferred_element_type=jnp.float32) + bx1
    h3 = t3 * jax.nn.sigmoid(t3)
    p = jnp.sum(h3 * wx2, axis=1, keepdims=True) + bx2
    p_ref[...] = p.reshape(1, BE // 128, 128)


def _edge_mlp(g, ea, em, we2, wx1, wea, aux):
    return pl.pallas_call(
        _edge_body,
        grid=(E // BE,),
        in_specs=[pl.BlockSpec((BE, D), lambda i: (i, 0)),
                  pl.BlockSpec((BE, DE), lambda i: (i, 0)),
                  pl.BlockSpec((BE, 1), lambda i: (i, 0)),
                  pl.BlockSpec((D, D), lambda i: (0, 0)),
                  pl.BlockSpec((D, D), lambda i: (0, 0)),
                  pl.BlockSpec((DE, D), lambda i: (0, 0)),
                  pl.BlockSpec((8, D), lambda i: (0, 0))],
        out_specs=[pl.BlockSpec((BE, D), lambda i: (i, 0)),
                   pl.BlockSpec((1, BE // 128, 128), lambda i: (i, 0, 0))],
        out_shape=[jax.ShapeDtypeStruct((E, D), jnp.float32),
                   jax.ShapeDtypeStruct((E // BE, BE // 128, 128), jnp.float32)],
    )(g, ea, em, we2, wx1, wea, aux)


def _sc_scatter_body(s_hbm, m_hbm, xd_hbm, p_hbm, mi_hbm, xa_hbm,
                     zbuf, zbufx,
                     sidx0, mbuf0, xdbuf0, pbuf0, sidx1, mbuf1, xdbuf1, pbuf1,
                     accm, accx,
                     lsem0, lsem1, ssem0, ssem1):
    c = lax.axis_index("c")
    s = lax.axis_index("s")
    wid = c * NS + s
    base0 = wid * EPW

    def zrow(e, cr):
        for j in range(D // 16):
            zbuf[e, pl.ds(j * 16, 16)] = jnp.zeros((16,), jnp.float32)
        zbufx[e, :] = jnp.zeros((16,), jnp.float32)
        return cr

    lax.fori_loop(0, ZR, zrow, 0)

    def zcp(t, cr):
        rb = s * ROWS + t * ZR
        pltpu.sync_copy(zbuf, accm.at[pl.ds(rb, ZR)])
        pltpu.sync_copy(zbufx, accx.at[pl.ds(rb, ZR)])
        return cr

    lax.fori_loop(0, ROWS // ZR, zcp, 0)
    plsc.subcore_barrier()

    iota16 = lax.iota(jnp.int32, 16)
    e3v = jnp.where(iota16 == 3, 1.0, 0.0).astype(jnp.float32)

    sets = ((sidx0, mbuf0, xdbuf0, pbuf0, lsem0, ssem0),
            (sidx1, mbuf1, xdbuf1, pbuf1, lsem1, ssem1))

    def start_loads(i, b):
        si, mb, xb, pb, ls, _ = sets[b]
        base = pl.multiple_of(base0 + i * KC, 8)
        pltpu.async_copy(s_hbm.at[pl.ds(base, KC)], si, ls)
        pltpu.async_copy(m_hbm.at[pl.ds(base, KC)], mb, ls)
        pltpu.async_copy(xd_hbm.at[pl.ds(base, KC)], xb, ls)
        pltpu.async_copy(p_hbm.at[pl.ds(base, KC)], pb, ls)

    def wait_loads(b):
        si, mb, xb, pb, ls, _ = sets[b]
        pltpu.make_async_copy(s_hbm.at[pl.ds(0, KC)], si, ls).wait()
        pltpu.make_async_copy(m_hbm.at[pl.ds(0, KC)], mb, ls).wait()
        pltpu.make_async_copy(xd_hbm.at[pl.ds(0, KC)], xb, ls).wait()
        pltpu.make_async_copy(p_hbm.at[pl.ds(0, KC)], pb, ls).wait()

    def compute(i, b):
        _, _, xb, pb, _, _ = sets[b]

        def grp(g, cr):
            pv = pb[pl.ds(g * 16, 16)]
            for l in range(16):
                e = g * 16 + l
                xb[e, :] = xb[e, :] * pv[l] + e3v
            return cr

        lax.fori_loop(0, KC // 16, grp, 0)

    def start_scatters(b):
        si, mb, xb, _, _, ss = sets[b]
        pltpu.async_copy(mb, accm.at[si], ss, add=True)
        pltpu.async_copy(xb, accx.at[si], ss, add=True)

    def wait_scatters(b):
        si, mb, xb, _, _, ss = sets[b]
        pltpu.make_async_copy(mb, accm.at[si], ss).wait()
        pltpu.make_async_copy(xb, accx.at[si], ss).wait()

    start_loads(0, 0)

    def pair(t, carry):
        wait_loads(0)

        @pl.when(t > 0)
        def _():
            wait_scatters(1)

        start_loads(2 * t + 1, 1)
        compute(2 * t, 0)
        start_scatters(0)

        wait_loads(1)

        @pl.when(t < NF // 2 - 1)
        def _():
            wait_scatters(0)
            start_loads(2 * t + 2, 0)

        compute(2 * t + 1, 1)
        start_scatters(1)
        return carry

    lax.fori_loop(0, NF // 2, pair, 0)

    si, mb, xb, pb, ls, ss = sets[0]
    toff = NF * KC
    tb = pl.multiple_of(base0 + toff, 8)
    wait_scatters(0)
    pltpu.async_copy(s_hbm.at[pl.ds(tb, KT)], si.at[pl.ds(0, KT)], ls)
    pltpu.async_copy(m_hbm.at[pl.ds(tb, KT)], mb.at[pl.ds(0, KT)], ls)
    pltpu.async_copy(xd_hbm.at[pl.ds(tb, KT)], xb.at[pl.ds(0, KT)], ls)
    pltpu.async_copy(p_hbm.at[pl.ds(tb, KT)], pb.at[pl.ds(0, KT)], ls)
    pltpu.make_async_copy(s_hbm.at[pl.ds(0, KT)], si.at[pl.ds(0, KT)], ls).wait()
    pltpu.make_async_copy(m_hbm.at[pl.ds(0, KT)], mb.at[pl.ds(0, KT)], ls).wait()
    pltpu.make_async_copy(xd_hbm.at[pl.ds(0, KT)], xb.at[pl.ds(0, KT)], ls).wait()
    pltpu.make_async_copy(p_hbm.at[pl.ds(0, KT)], pb.at[pl.ds(0, KT)], ls).wait()
    pv = pb[pl.ds(0, KT)]
    for l in range(KT):
        xb[l, :] = xb[l, :] * pv[l] + e3v
    sv = si[pl.ds(0, KT)]
    pltpu.async_copy(mb.at[pl.ds(0, KT)], accm.at[sv], ss, add=True)
    pltpu.async_copy(xb.at[pl.ds(0, KT)], accx.at[sv], ss, add=True)
    wait_scatters(1)
    pltpu.make_async_copy(mb.at[pl.ds(0, KT)], accm.at[sv], ss).wait()
    pltpu.make_async_copy(xb.at[pl.ds(0, KT)], accx.at[sv], ss).wait()
    plsc.subcore_barrier()

    rb = s * ROWS
    pltpu.sync_copy(accm.at[pl.ds(rb, ROWS)], mi_hbm.at[c, pl.ds(rb, ROWS)])
    pltpu.sync_copy(accx.at[pl.ds(rb, ROWS)], xa_hbm.at[c, pl.ds(rb, ROWS)])


_sc_scatter = functools.partial(
    pl.kernel,
    mesh=_mesh,
    out_type=[jax.ShapeDtypeStruct((NC, N, D), jnp.float32),
              jax.ShapeDtypeStruct((NC, N, 16), jnp.float32)],
    scratch_types=[pltpu.VMEM((ZR, D), jnp.float32),
                   pltpu.VMEM((ZR, 16), jnp.float32),
                   pltpu.VMEM((KC,), jnp.int32),
                   pltpu.VMEM((KC, D), jnp.float32),
                   pltpu.VMEM((KC, 16), jnp.float32),
                   pltpu.VMEM((KC,), jnp.float32),
                   pltpu.VMEM((KC,), jnp.int32),
                   pltpu.VMEM((KC, D), jnp.float32),
                   pltpu.VMEM((KC, 16), jnp.float32),
                   pltpu.VMEM((KC,), jnp.float32),
                   pltpu.VMEM_SHARED((N, D), jnp.float32),
                   pltpu.VMEM_SHARED((N, 16), jnp.float32),
                   pltpu.SemaphoreType.DMA,
                   pltpu.SemaphoreType.DMA,
                   pltpu.SemaphoreType.DMA,
                   pltpu.SemaphoreType.DMA],
    compiler_params=_sc_params,
)(_sc_scatter_body)


def _node_body(h_ref, hh_ref, mi_ref, xa_ref, xp_ref, nm_ref, wh1b_ref,
               wh2_ref, aux_ref, hn_ref, co_ref):
    aux = aux_ref[...]
    bh1 = aux[0:1, :]
    bh2 = aux[1:2, :]
    m2 = mi_ref[...]
    mi = m2[0] + m2[1]
    x2 = xa_ref[...]
    xa = x2[0] + x2[1]
    nm = nm_ref[...]
    t = hh_ref[...] + jnp.dot(mi, wh1b_ref[...],
                              preferred_element_type=jnp.float32) + bh1
    u = t * jax.nn.sigmoid(t)
    hn = h_ref[...] + jnp.dot(u, wh2_ref[...],
                              preferred_element_type=jnp.float32) + bh2
    hn_ref[...] = hn * nm
    cnt = xa[:, 3:4]
    mean = xa[:, 0:3] / cnt
    co_ref[...] = (xp_ref[...][:, 0:3] + mean) * nm


def _node_update(h, hh, mi, xa, xp, nm, wh1b, wh2, aux2):
    bn = 2000
    blk = pl.BlockSpec((bn, D), lambda i: (i, 0))
    return pl.pallas_call(
        _node_body,
        grid=(N // bn,),
        in_specs=[blk, blk,
                  pl.BlockSpec((NC, bn, D), lambda i: (0, i, 0)),
                  pl.BlockSpec((NC, bn, 16), lambda i: (0, i, 0)),
                  pl.BlockSpec((bn, 16), lambda i: (i, 0)),
                  pl.BlockSpec((bn, 1), lambda i: (i, 0)),
                  pl.BlockSpec((D, D), lambda i: (0, 0)),
                  pl.BlockSpec((D, D), lambda i: (0, 0)),
                  pl.BlockSpec((2, D), lambda i: (0, 0))],
        out_specs=[blk, pl.BlockSpec((bn, 3), lambda i: (i, 0))],
        out_shape=[jax.ShapeDtypeStruct((N, D), jnp.float32),
                   jax.ShapeDtypeStruct((N, 3), jnp.float32)],
    )(h, hh, mi, xa, xp, nm, wh1b, wh2, aux2)


def kernel(edge_index, h, x, edge_attr, node_mask, edge_mask,
           W_e1, b_e1, W_e2, b_e2, W_att, b_att,
           W_h1, b_h1, W_h2, b_h2, W_x1, b_x1, W_x2, b_x2):
    w1hi = W_e1[:D]
    w1hj = W_e1[D:2 * D]
    w1r = W_e1[2 * D]
    wea = W_e1[2 * D + 1:]
    wh1a = W_h1[:D]
    wh1b = W_h1[D:]
    senders = edge_index[0]
    receivers = edge_index[1]
    xpad = jnp.pad(x, ((0, 0), (0, 13)))

    scal = jnp.zeros((D,), jnp.float32).at[0].set(b_att[0]).at[1].set(b_x2[0])
    aux = jnp.stack([w1r, b_e1, b_e2, b_x1, scal,
                     W_att[:, 0], W_x2[:, 0], jnp.zeros((D,), jnp.float32)])
    aux2 = jnp.stack([b_h1, b_h2])

    hs, hr, hh = _precompute(h, w1hi, w1hj, wh1a)
    g, xd = _sc_gather(hs, hr, xpad, senders, receivers, w1r)
    m, p2d = _edge_mlp(g, edge_attr, edge_mask, W_e2, W_x1, wea, aux)
    pflat = p2d.reshape(E)
    mi, xa = _sc_scatter(senders, m, xd, pflat)
    h_new, coord = _node_update(h, hh, mi, xa, xpad, node_mask, wh1b, W_h2, aux2)
    return (h_new, coord)


# speedup vs baseline: 5.0974x; 1.0425x over previous
import functools
import jax
import jax.numpy as jnp
from jax import lax
from jax.experimental import pallas as pl
from jax.experimental.pallas import tpu as pltpu
from jax.experimental.pallas import tpu_sc as plsc

N = 10000
E = 320000
D = 128
DE = 16

NC = 2
NS = 16
NW = NC * NS
EPW = E // NW
KC = 128
NF = EPW // KC
KT = EPW - NF * KC
ROWS = N // NS
ZR = 25

_mesh = plsc.VectorSubcoreMesh(core_axis_name="c", subcore_axis_name="s")
_sc_params = pltpu.CompilerParams(use_tc_tiling_on_sc=False)


def _pre_body(h_ref, wi_ref, wj_ref, wh_ref, hs_ref, hr_ref, hh_ref):
    hb = h_ref[...]
    hs_ref[...] = jnp.dot(hb, wi_ref[...], preferred_element_type=jnp.float32)
    hr_ref[...] = jnp.dot(hb, wj_ref[...], preferred_element_type=jnp.float32)
    hh_ref[...] = jnp.dot(hb, wh_ref[...], preferred_element_type=jnp.float32)


def _precompute(h, w1hi, w1hj, wh1a):
    bn = 2000
    blk = pl.BlockSpec((bn, D), lambda i: (i, 0))
    wblk = pl.BlockSpec((D, D), lambda i: (0, 0))
    out = jax.ShapeDtypeStruct((N, D), jnp.float32)
    return pl.pallas_call(
        _pre_body,
        grid=(N // bn,),
        in_specs=[blk, wblk, wblk, wblk],
        out_specs=[blk, blk, blk],
        out_shape=[out, out, out],
    )(h, w1hi, w1hj, wh1a)


def _sc_gather_body(hs_hbm, hr_hbm, xp_hbm, s_hbm, r_hbm,
                    w1r_hbm, g_hbm, xd_hbm,
                    sall, rall, w1rv,
                    bufa0, bufb0, bufxs0, bufxr0,
                    bufa1, bufb1, bufxs1, bufxr1,
                    bufa2, bufb2, bufxs2, bufxr2,
                    gsem0, gsem1, gsem2, wsem0, wsem1, wsem2):
    c = lax.axis_index("c")
    s = lax.axis_index("s")
    wid = c * NS + s
    base0 = wid * EPW
    pltpu.sync_copy(w1r_hbm, w1rv)
    pltpu.sync_copy(s_hbm.at[pl.ds(base0, EPW)], sall)
    pltpu.sync_copy(r_hbm.at[pl.ds(base0, EPW)], rall)

    sets = ((bufa0, bufb0, bufxs0, bufxr0, gsem0, wsem0),
            (bufa1, bufb1, bufxs1, bufxr1, gsem1, wsem1),
            (bufa2, bufb2, bufxs2, bufxr2, gsem2, wsem2))

    def start_gathers(i, b):
        ba, bb, bxs, bxr, gs, _ = sets[b]
        off = pl.multiple_of(i * KC, 8)
        si = sall.at[pl.ds(off, KC)]
        ri = rall.at[pl.ds(off, KC)]
        pltpu.async_copy(hs_hbm.at[si], ba, gs)
        pltpu.async_copy(hr_hbm.at[ri], bb, gs)
        pltpu.async_copy(xp_hbm.at[si], bxs, gs)
        pltpu.async_copy(xp_hbm.at[ri], bxr, gs)

    def wait_gathers(b):
        ba, bb, bxs, bxr, gs, _ = sets[b]
        pltpu.make_async_copy(hs_hbm.at[pl.ds(0, KC)], ba, gs).wait()
        pltpu.make_async_copy(hr_hbm.at[pl.ds(0, KC)], bb, gs).wait()
        pltpu.make_async_copy(xp_hbm.at[pl.ds(0, KC)], bxs, gs).wait()
        pltpu.make_async_copy(xp_hbm.at[pl.ds(0, KC)], bxr, gs).wait()

    def compute(b, nrows):
        ba, bb, bxs, bxr, _, _ = sets[b]

        def row(e, cr):
            v = bxs[e, :] - bxr[e, :]
            bxs[e, :] = v
            sq = v * v
            rad = sq[0] + sq[1] + sq[2]
            for j in range(D // 16):
                sl = pl.ds(j * 16, 16)
                ba[e, sl] = ba[e, sl] + bb[e, sl] + rad * w1rv[sl]
            return cr

        lax.fori_loop(0, nrows, row, 0)

    def start_writes(i, b):
        ba, _, bxs, _, _, ws = sets[b]
        base = pl.multiple_of(base0 + i * KC, 8)
        pltpu.async_copy(ba, g_hbm.at[pl.ds(base, KC)], ws)
        pltpu.async_copy(bxs, xd_hbm.at[pl.ds(base, KC)], ws)

    def wait_writes(b):
        ba, _, bxs, _, _, ws = sets[b]
        pltpu.make_async_copy(ba, g_hbm.at[pl.ds(0, KC)], ws).wait()
        pltpu.make_async_copy(bxs, xd_hbm.at[pl.ds(0, KC)], ws).wait()

    start_gathers(0, 0)
    start_gathers(1, 1)

    NT = NF // 3

    def triple(t, carry):
        wait_gathers(0)
        compute(0, KC)
        start_writes(3 * t, 0)

        @pl.when(t > 0)
        def _():
            wait_writes(2)

        start_gathers(3 * t + 2, 2)

        wait_gathers(1)
        compute(1, KC)
        start_writes(3 * t + 1, 1)

        @pl.when(t < NT - 1)
        def _():
            wait_writes(0)
            start_gathers(3 * t + 3, 0)

        wait_gathers(2)
        compute(2, KC)
        start_writes(3 * t + 2, 2)

        @pl.when(t < NT - 1)
        def _():
            wait_writes(1)
            start_gathers(3 * t + 4, 1)

        return carry

    lax.fori_loop(0, NT, triple, 0)

    ba, bb, bxs, bxr, gs, ws = sets[0]
    toff = NF * KC
    sv = sall[pl.ds(toff, KT)]
    rv = rall[pl.ds(toff, KT)]
    wait_writes(0)
    pltpu.async_copy(hs_hbm.at[sv], ba.at[pl.ds(0, KT)], gs)
    pltpu.async_copy(hr_hbm.at[rv], bb.at[pl.ds(0, KT)], gs)
    pltpu.async_copy(xp_hbm.at[sv], bxs.at[pl.ds(0, KT)], gs)
    pltpu.async_copy(xp_hbm.at[rv], bxr.at[pl.ds(0, KT)], gs)
    pltpu.make_async_copy(hs_hbm.at[pl.ds(0, KT)], ba.at[pl.ds(0, KT)], gs).wait()
    pltpu.make_async_copy(hr_hbm.at[pl.ds(0, KT)], bb.at[pl.ds(0, KT)], gs).wait()
    pltpu.make_async_copy(xp_hbm.at[pl.ds(0, KT)], bxs.at[pl.ds(0, KT)], gs).wait()
    pltpu.make_async_copy(xp_hbm.at[pl.ds(0, KT)], bxr.at[pl.ds(0, KT)], gs).wait()
    compute(0, KT)
    tb = pl.multiple_of(base0 + toff, 8)
    pltpu.async_copy(ba.at[pl.ds(0, KT)], g_hbm.at[pl.ds(tb, KT)], ws)
    pltpu.async_copy(bxs.at[pl.ds(0, KT)], xd_hbm.at[pl.ds(tb, KT)], ws)
    wait_writes(1)
    wait_writes(2)
    pltpu.make_async_copy(ba.at[pl.ds(0, KT)], g_hbm.at[pl.ds(0, KT)], ws).wait()
    pltpu.make_async_copy(bxs.at[pl.ds(0, KT)], xd_hbm.at[pl.ds(0, KT)], ws).wait()


_sc_gather = functools.partial(
    pl.kernel,
    mesh=_mesh,
    out_type=[jax.ShapeDtypeStruct((E, D), jnp.float32),
              jax.ShapeDtypeStruct((E, 16), jnp.float32)],
    scratch_types=[pltpu.VMEM((EPW,), jnp.int32),
                   pltpu.VMEM((EPW,), jnp.int32),
                   pltpu.VMEM((D,), jnp.float32)]
    + [pltpu.VMEM((KC, D), jnp.float32),
       pltpu.VMEM((KC, D), jnp.float32),
       pltpu.VMEM((KC, 16), jnp.float32),
       pltpu.VMEM((KC, 16), jnp.float32)] * 3
    + [pltpu.SemaphoreType.DMA] * 6,
    compiler_params=_sc_params,
)(_sc_gather_body)


BE = 2560


def _edge_body(g_ref, ea_ref, em_ref, we2_ref, wx1_ref, wea_ref,
               aux_ref, m_ref, p_ref):
    g = g_ref[...]
    aux = aux_ref[...]
    be1 = aux[1:2, :]
    be2 = aux[2:3, :]
    bx1 = aux[3:4, :]
    watt = aux[5:6, :]
    wx2 = aux[6:7, :]
    batt = aux[4, 0]
    bx2 = aux[4, 1]

    t1 = g + jnp.dot(ea_ref[...], wea_ref[...],
                     preferred_element_type=jnp.float32) + be1
    o1 = t1 * jax.nn.sigmoid(t1)
    t2 = jnp.dot(o1, we2_ref[...], preferred_element_type=jnp.float32) + be2
    o2 = t2 * jax.nn.sigmoid(t2)
    att = jax.nn.sigmoid(jnp.sum(o2 * watt, axis=1, keepdims=True) + batt)
    m = o2 * att * em_ref[...]
    m_ref[...] = m
    t3 = jnp.dot(m, wx1_ref[...], preferred_element_type=jnp.float32) + bx1
    h3 = t3 * jax.nn.sigmoid(t3)
    p = jnp.sum(h3 * wx2, axis=1, keepdims=True) + bx2
    p_ref[...] = p.reshape(1, BE // 128, 128)


def _edge_mlp(g, ea, em, we2, wx1, wea, aux):
    return pl.pallas_call(
        _edge_body,
        grid=(E // BE,),
        in_specs=[pl.BlockSpec((BE, D), lambda i: (i, 0)),
                  pl.BlockSpec((BE, DE), lambda i: (i, 0)),
                  pl.BlockSpec((BE, 1), lambda i: (i, 0)),
                  pl.BlockSpec((D, D), lambda i: (0, 0)),
                  pl.BlockSpec((D, D), lambda i: (0, 0)),
                  pl.BlockSpec((DE, D), lambda i: (0, 0)),
                  pl.BlockSpec((8, D), lambda i: (0, 0))],
        out_specs=[pl.BlockSpec((BE, D), lambda i: (i, 0)),
                   pl.BlockSpec((1, BE // 128, 128), lambda i: (i, 0, 0))],
        out_shape=[jax.ShapeDtypeStruct((E, D), jnp.float32),
                   jax.ShapeDtypeStruct((E // BE, BE // 128, 128), jnp.float32)],
    )(g, ea, em, we2, wx1, wea, aux)


def _sc_scatter_body(s_hbm, m_hbm, xd_hbm, p_hbm, mi_hbm, xa_hbm,
                     zbuf, zbufx,
                     sidx0, mbuf0, xdbuf0, pbuf0, sidx1, mbuf1, xdbuf1, pbuf1,
                     accm, accx,
                     lsem0, lsem1, ssem0, ssem1):
    c = lax.axis_index("c")
    s = lax.axis_index("s")
    wid = c * NS + s
    base0 = wid * EPW

    def zrow(e, cr):
        for j in range(D // 16):
            zbuf[e, pl.ds(j * 16, 16)] = jnp.zeros((16,), jnp.float32)
        zbufx[e, :] = jnp.zeros((16,), jnp.float32)
        return cr

    lax.fori_loop(0, ZR, zrow, 0)

    def zcp(t, cr):
        rb = s * ROWS + t * ZR
        pltpu.sync_copy(zbuf, accm.at[pl.ds(rb, ZR)])
        pltpu.sync_copy(zbufx, accx.at[pl.ds(rb, ZR)])
        return cr

    lax.fori_loop(0, ROWS // ZR, zcp, 0)
    plsc.subcore_barrier()

    iota16 = lax.iota(jnp.int32, 16)
    e3v = jnp.where(iota16 == 3, 1.0, 0.0).astype(jnp.float32)

    sets = ((sidx0, mbuf0, xdbuf0, pbuf0, lsem0, ssem0),
            (sidx1, mbuf1, xdbuf1, pbuf1, lsem1, ssem1))

    def start_loads(i, b):
        si, mb, xb, pb, ls, _ = sets[b]
        base = pl.multiple_of(base0 + i * KC, 8)
        pltpu.async_copy(s_hbm.at[pl.ds(base, KC)], si, ls)
        pltpu.async_copy(m_hbm.at[pl.ds(base, KC)], mb, ls)
        pltpu.async_copy(xd_hbm.at[pl.ds(base, KC)], xb, ls)
        pltpu.async_copy(p_hbm.at[pl.ds(base, KC)], pb, ls)

    def wait_loads(b):
        si, mb, xb, pb, ls, _ = sets[b]
        pltpu.make_async_copy(s_hbm.at[pl.ds(0, KC)], si, ls).wait()
        pltpu.make_async_copy(m_hbm.at[pl.ds(0, KC)], mb, ls).wait()
        pltpu.make_async_copy(xd_hbm.at[pl.ds(0, KC)], xb, ls).wait()
        pltpu.make_async_copy(p_hbm.at[pl.ds(0, KC)], pb, ls).wait()

    def compute(i, b):
        _, _, xb, pb, _, _ = sets[b]

        def grp(g, cr):
            pv = pb[pl.ds(g * 16, 16)]
            for l in range(16):
                e = g * 16 + l
                xb[e, :] = xb[e, :] * pv[l] + e3v
            return cr

        lax.fori_loop(0, KC // 16, grp, 0)

    def start_scatters(b):
        si, mb, xb, _, _, ss = sets[b]
        pltpu.async_copy(mb, accm.at[si], ss, add=True)
        pltpu.async_copy(xb, accx.at[si], ss, add=True)

    def wait_scatters(b):
        si, mb, xb, _, _, ss = sets[b]
        pltpu.make_async_copy(mb, accm.at[si], ss).wait()
        pltpu.make_async_copy(xb, accx.at[si], ss).wait()

    start_loads(0, 0)

    def pair(t, carry):
        wait_loads(0)

        @pl.when(t > 0)
        def _():
            wait_scatters(1)

        start_loads(2 * t + 1, 1)
        compute(2 * t, 0)
        start_scatters(0)

        wait_loads(1)

        @pl.when(t < NF // 2 - 1)
        def _():
            wait_scatters(0)
            start_loads(2 * t + 2, 0)

        compute(2 * t + 1, 1)
        start_scatters(1)
        return carry

    lax.fori_loop(0, NF // 2, pair, 0)

    si, mb, xb, pb, ls, ss = sets[0]
    toff = NF * KC
    tb = pl.multiple_of(base0 + toff, 8)
    wait_scatters(0)
    pltpu.async_copy(s_hbm.at[pl.ds(tb, KT)], si.at[pl.ds(0, KT)], ls)
    pltpu.async_copy(m_hbm.at[pl.ds(tb, KT)], mb.at[pl.ds(0, KT)], ls)
    pltpu.async_copy(xd_hbm.at[pl.ds(tb, KT)], xb.at[pl.ds(0, KT)], ls)
    pltpu.async_copy(p_hbm.at[pl.ds(tb, KT)], pb.at[pl.ds(0, KT)], ls)
    pltpu.make_async_copy(s_hbm.at[pl.ds(0, KT)], si.at[pl.ds(0, KT)], ls).wait()
    pltpu.make_async_copy(m_hbm.at[pl.ds(0, KT)], mb.at[pl.ds(0, KT)], ls).wait()
    pltpu.make_async_copy(xd_hbm.at[pl.ds(0, KT)], xb.at[pl.ds(0, KT)], ls).wait()
    pltpu.make_async_copy(p_hbm.at[pl.ds(0, KT)], pb.at[pl.ds(0, KT)], ls).wait()
    pv = pb[pl.ds(0, KT)]
    for l in range(KT):
        xb[l, :] = xb[l, :] * pv[l] + e3v
    sv = si[pl.ds(0, KT)]
    pltpu.async_copy(mb.at[pl.ds(0, KT)], accm.at[sv], ss, add=True)
    pltpu.async_copy(xb.at[pl.ds(0, KT)], accx.at[sv], ss, add=True)
    wait_scatters(1)
    pltpu.make_async_copy(mb.at[pl.ds(0, KT)], accm.at[sv], ss).wait()
    pltpu.make_async_copy(xb.at[pl.ds(0, KT)], accx.at[sv], ss).wait()
    plsc.subcore_barrier()

    rb = s * ROWS
    pltpu.sync_copy(accm.at[pl.ds(rb, ROWS)], mi_hbm.at[c, pl.ds(rb, ROWS)])
    pltpu.sync_copy(accx.at[pl.ds(rb, ROWS)], xa_hbm.at[c, pl.ds(rb, ROWS)])


_sc_scatter = functools.partial(
    pl.kernel,
    mesh=_mesh,
    out_type=[jax.ShapeDtypeStruct((NC, N, D), jnp.float32),
              jax.ShapeDtypeStruct((NC, N, 16), jnp.float32)],
    scratch_types=[pltpu.VMEM((ZR, D), jnp.float32),
                   pltpu.VMEM((ZR, 16), jnp.float32),
                   pltpu.VMEM((KC,), jnp.int32),
                   pltpu.VMEM((KC, D), jnp.float32),
                   pltpu.VMEM((KC, 16), jnp.float32),
                   pltpu.VMEM((KC,), jnp.float32),
                   pltpu.VMEM((KC,), jnp.int32),
                   pltpu.VMEM((KC, D), jnp.float32),
                   pltpu.VMEM((KC, 16), jnp.float32),
                   pltpu.VMEM((KC,), jnp.float32),
                   pltpu.VMEM_SHARED((N, D), jnp.float32),
                   pltpu.VMEM_SHARED((N, 16), jnp.float32),
                   pltpu.SemaphoreType.DMA,
                   pltpu.SemaphoreType.DMA,
                   pltpu.SemaphoreType.DMA,
                   pltpu.SemaphoreType.DMA],
    compiler_params=_sc_params,
)(_sc_scatter_body)


def _node_body(h_ref, hh_ref, mi_ref, xa_ref, xp_ref, nm_ref, wh1b_ref,
               wh2_ref, aux_ref, hn_ref, co_ref):
    aux = aux_ref[...]
    bh1 = aux[0:1, :]
    bh2 = aux[1:2, :]
    m2 = mi_ref[...]
    mi = m2[0] + m2[1]
    x2 = xa_ref[...]
    xa = x2[0] + x2[1]
    nm = nm_ref[...]
    t = hh_ref[...] + jnp.dot(mi, wh1b_ref[...],
                              preferred_element_type=jnp.float32) + bh1
    u = t * jax.nn.sigmoid(t)
    hn = h_ref[...] + jnp.dot(u, wh2_ref[...],
                              preferred_element_type=jnp.float32) + bh2
    hn_ref[...] = hn * nm
    cnt = xa[:, 3:4]
    mean = xa[:, 0:3] / cnt
    co_ref[...] = (xp_ref[...][:, 0:3] + mean) * nm


def _node_update(h, hh, mi, xa, xp, nm, wh1b, wh2, aux2):
    bn = 2000
    blk = pl.BlockSpec((bn, D), lambda i: (i, 0))
    return pl.pallas_call(
        _node_body,
        grid=(N // bn,),
        in_specs=[blk, blk,
                  pl.BlockSpec((NC, bn, D), lambda i: (0, i, 0)),
                  pl.BlockSpec((NC, bn, 16), lambda i: (0, i, 0)),
                  pl.BlockSpec((bn, 16), lambda i: (i, 0)),
                  pl.BlockSpec((bn, 1), lambda i: (i, 0)),
                  pl.BlockSpec((D, D), lambda i: (0, 0)),
                  pl.BlockSpec((D, D), lambda i: (0, 0)),
                  pl.BlockSpec((2, D), lambda i: (0, 0))],
        out_specs=[blk, pl.BlockSpec((bn, 3), lambda i: (i, 0))],
        out_shape=[jax.ShapeDtypeStruct((N, D), jnp.float32),
                   jax.ShapeDtypeStruct((N, 3), jnp.float32)],
    )(h, hh, mi, xa, xp, nm, wh1b, wh2, aux2)


def kernel(edge_index, h, x, edge_attr, node_mask, edge_mask,
           W_e1, b_e1, W_e2, b_e2, W_att, b_att,
           W_h1, b_h1, W_h2, b_h2, W_x1, b_x1, W_x2, b_x2):
    w1hi = W_e1[:D]
    w1hj = W_e1[D:2 * D]
    w1r = W_e1[2 * D]
    wea = W_e1[2 * D + 1:]
    wh1a = W_h1[:D]
    wh1b = W_h1[D:]
    senders = edge_index[0]
    receivers = edge_index[1]
    xpad = jnp.pad(x, ((0, 0), (0, 13)))

    scal = jnp.zeros((D,), jnp.float32).at[0].set(b_att[0]).at[1].set(b_x2[0])
    aux = jnp.stack([w1r, b_e1, b_e2, b_x1, scal,
                     W_att[:, 0], W_x2[:, 0], jnp.zeros((D,), jnp.float32)])
    aux2 = jnp.stack([b_h1, b_h2])

    hs, hr, hh = _precompute(h, w1hi, w1hj, wh1a)
    g, xd = _sc_gather(hs, hr, xpad, senders, receivers, w1r)
    m, p2d = _edge_mlp(g, edge_attr, edge_mask, W_e2, W_x1, wea, aux)
    pflat = p2d.reshape(E)
    mi, xa = _sc_scatter(senders, m, xd, pflat)
    h_new, coord = _node_update(h, hh, mi, xa, xpad, node_mask, wh1b, W_h2, aux2)
    return (h_new, coord)
